# Initial kernel scaffold; baseline (speedup 1.0000x reference)
import jax
import jax.numpy as jnp
from jax.experimental import pallas as pl


def kernel(embed_on_edges, node_attr, edge_src, edge_dst, edge_attr, edge_length_embedding, batch, params_edges, params_final):
    raise NotImplementedError("write your pallas kernel here")


# speedup vs baseline: 2.2912x; 2.2912x over previous
import functools

import jax
import jax.numpy as jnp
import numpy as np
from jax import lax
from jax.experimental import pallas as pl
from jax.experimental.pallas import tpu as pltpu
from jax.experimental.pallas import tpu_sc as plsc

N = 50000
E = 800000
NG = 64
D = 64
DH = 32
NPAD = 51200
BN = 512
BE = 2000
CH = 400
NTILES = 16
EDGES_PER_TILE = E // NTILES
ROWS_PER_TILE = NPAD // NTILES
INV_SQRT_NEIGH = 0.25


def _node_body(x_ref, na_ref, wm_ref, wsx_ref, wsa_ref, y2_ref, sc_ref):
    xb = x_ref[...]
    y = jnp.dot(xb, wm_ref[...], preferred_element_type=jnp.float32)
    y2_ref[0] = y[:, :DH]
    y2_ref[1] = y[:, DH:]
    sc = jnp.dot(xb, wsx_ref[...], preferred_element_type=jnp.float32)
    sc += jnp.dot(na_ref[...], wsa_ref[...], preferred_element_type=jnp.float32)
    sc_ref[...] = sc


def _node_transform(x, na, wm, wsx, wsa):
    grid = (NPAD // BN,)
    return pl.pallas_call(
        _node_body,
        grid=grid,
        in_specs=[
            pl.BlockSpec((BN, D), lambda i: (i, 0)),
            pl.BlockSpec((BN, 8), lambda i: (i, 0)),
            pl.BlockSpec((D, D), lambda i: (0, 0)),
            pl.BlockSpec((D, D), lambda i: (0, 0)),
            pl.BlockSpec((8, D), lambda i: (0, 0)),
        ],
        out_specs=[
            pl.BlockSpec((2, BN, DH), lambda i: (0, i, 0)),
            pl.BlockSpec((BN, D), lambda i: (i, 0)),
        ],
        out_shape=[
            jax.ShapeDtypeStruct((2, NPAD, DH), jnp.float32),
            jax.ShapeDtypeStruct((NPAD, D), jnp.float32),
        ],
    )(x, na, wm, wsx, wsa)


def _edge_body(elen_ref, eattr_ref, fc1_ref, fc2_ref, we_ref, coef_ref):
    w8 = jnp.maximum(
        jnp.dot(elen_ref[...], fc1_ref[...], preferred_element_type=jnp.float32),
        0.0)
    w = jnp.dot(w8, fc2_ref[...], preferred_element_type=jnp.float32)
    ew = jnp.dot(eattr_ref[...], we_ref[...], preferred_element_type=jnp.float32)
    cf = w * ew
    coef_ref[0] = cf[:, :DH]
    coef_ref[1] = cf[:, DH:]


def _edge_coef(elen, eattr, fc1, fc2, we):
    grid = (E // BE,)
    return pl.pallas_call(
        _edge_body,
        grid=grid,
        in_specs=[
            pl.BlockSpec((BE, 10), lambda i: (i, 0)),
            pl.BlockSpec((BE, 9), lambda i: (i, 0)),
            pl.BlockSpec((10, 8), lambda i: (0, 0)),
            pl.BlockSpec((8, D), lambda i: (0, 0)),
            pl.BlockSpec((9, D), lambda i: (0, 0)),
        ],
        out_specs=pl.BlockSpec((2, BE, DH), lambda i: (0, i, 0)),
        out_shape=jax.ShapeDtypeStruct((2, E, DH), jnp.float32),
    )(elen, eattr, fc1, fc2, we)


def _combine_body(act, sc_ref, agg_ref, out_ref):
    agg = jnp.concatenate([agg_ref[0], agg_ref[1]], axis=1)
    h = sc_ref[...] + agg * INV_SQRT_NEIGH
    out_ref[...] = jax.nn.gelu(h) if act else h


def _combine(sc, agg, act):
    grid = (NPAD // BN,)
    return pl.pallas_call(
        functools.partial(_combine_body, act),
        grid=grid,
        in_specs=[
            pl.BlockSpec((BN, D), lambda i: (i, 0)),
            pl.BlockSpec((2, BN, DH), lambda i: (0, i, 0)),
        ],
        out_specs=pl.BlockSpec((BN, D), lambda i: (i, 0)),
        out_shape=jax.ShapeDtypeStruct((NPAD, D), jnp.float32),
    )(sc, agg)


def _pool_body(x_ref, b_ref, g_ref, y_ref, acc_ref):
    i = pl.program_id(0)

    @pl.when(i == 0)
    def _():
        acc_ref[...] = jnp.zeros((NG, D), jnp.float32)

    b = b_ref[0, 0, :]
    oh = (lax.broadcasted_iota(jnp.int32, (NG, BN), 0) == b[None, :])
    acc_ref[...] += jnp.dot(oh.astype(jnp.float32), x_ref[...],
                            preferred_element_type=jnp.float32)

    @pl.when(i == pl.num_programs(0) - 1)
    def _():
        acc = acc_ref[...]
        z = (acc[:, :10] + g_ref[...]) * 0.01
        m = jnp.max(z, axis=1, keepdims=True)
        e = jnp.exp(z - m)
        y_ref[...] = e / jnp.sum(e, axis=1, keepdims=True)


def _pool(x, batch3, g):
    grid = (NPAD // BN,)
    return pl.pallas_call(
        _pool_body,
        grid=grid,
        in_specs=[
            pl.BlockSpec((BN, D), lambda i: (i, 0)),
            pl.BlockSpec((1, 1, BN), lambda i: (i, 0, 0)),
            pl.BlockSpec((NG, 10), lambda i: (0, 0)),
        ],
        out_specs=pl.BlockSpec((NG, 10), lambda i: (0, 0)),
        out_shape=jax.ShapeDtypeStruct((NG, 10), jnp.float32),
        scratch_shapes=[pltpu.VMEM((NG, D), jnp.float32)],
    )(x, batch3, g)


def _sc_body(y_hbm, coef_hbm, esrc_hbm, edst_hbm, out_hbm,
             src_v, dst_v, rows_v, coef_v, agg_sh, sem):
    c = lax.axis_index("c")
    s = lax.axis_index("s")

    def zrow(r, carry):
        rows_v[r, pl.ds(0, 16)] = jnp.zeros((16,), jnp.float32)
        rows_v[r, pl.ds(16, 16)] = jnp.zeros((16,), jnp.float32)
        return carry

    lax.fori_loop(0, CH, zrow, 0)
    base_r = s * ROWS_PER_TILE

    def zcp(k, carry):
        pltpu.sync_copy(rows_v, agg_sh.at[pl.ds(base_r + k * CH, CH)])
        return carry

    lax.fori_loop(0, ROWS_PER_TILE // CH, zcp, 0)
    plsc.subcore_barrier()

    ebase = s * EDGES_PER_TILE

    def chunk(i, carry):
        b = ebase + i * CH
        pltpu.sync_copy(esrc_hbm.at[pl.ds(b, CH)], src_v)
        pltpu.sync_copy(edst_hbm.at[pl.ds(b, CH)], dst_v)
        pltpu.sync_copy(coef_hbm.at[c].at[pl.ds(b, CH)], coef_v)
        pltpu.async_copy(y_hbm.at[c].at[src_v], rows_v, sem).wait()

        def mul(r, carry2):
            rows_v[r, pl.ds(0, 16)] = (rows_v[r, pl.ds(0, 16)]
                                       * coef_v[r, pl.ds(0, 16)])
            rows_v[r, pl.ds(16, 16)] = (rows_v[r, pl.ds(16, 16)]
                                        * coef_v[r, pl.ds(16, 16)])
            return carry2

        lax.fori_loop(0, CH, mul, 0)
        pltpu.sync_copy(rows_v, agg_sh.at[dst_v], add=True)
        return carry

    lax.fori_loop(0, EDGES_PER_TILE // CH, chunk, 0)
    plsc.subcore_barrier()

    def fcp(k, carry):
        off = base_r + k * CH
        pltpu.sync_copy(agg_sh.at[pl.ds(off, CH)],
                        out_hbm.at[c].at[pl.ds(off, CH)])
        return carry

    lax.fori_loop(0, ROWS_PER_TILE // CH, fcp, 0)


@functools.lru_cache(maxsize=None)
def _sc_edge_kernel():
    mesh = plsc.VectorSubcoreMesh(core_axis_name="c", subcore_axis_name="s")
    return pl.kernel(
        _sc_body,
        out_type=jax.ShapeDtypeStruct((2, NPAD, DH), jnp.float32),
        mesh=mesh,
        scratch_types=[
            pltpu.VMEM((CH,), jnp.int32),
            pltpu.VMEM((CH,), jnp.int32),
            pltpu.VMEM((CH, DH), jnp.float32),
            pltpu.VMEM((CH, DH), jnp.float32),
            pltpu.VMEM_SHARED((NPAD, DH), jnp.float32),
            pltpu.SemaphoreType.DMA,
        ],
        compiler_params=pltpu.CompilerParams(use_tc_tiling_on_sc=False),
    )


def _sc_edge(y2, coef, esrc, edst):
    return _sc_edge_kernel()(y2, coef, esrc, edst)


def _pad_w(w, rows, cols):
    return jnp.pad(w, ((0, rows - w.shape[0]), (0, cols - w.shape[1])))


def kernel(embed_on_edges, node_attr, edge_src, edge_dst, edge_attr,
           edge_length_embedding, batch, params_edges, params_final):
    x = jnp.pad(embed_on_edges,
                ((0, NPAD - N), (0, D - embed_on_edges.shape[1])))
    na = jnp.pad(node_attr, ((0, NPAD - N), (0, 0)))
    esrc = edge_src.astype(jnp.int32)
    edst = edge_dst.astype(jnp.int32)

    params = list(params_edges) + list(params_final)
    n_edge_layers = len(params_edges)
    d_ins = [embed_on_edges.shape[1]] + [36] * (len(params) - 1)

    for li, p in enumerate(params):
        d_in = d_ins[li]
        wm = _pad_w(p["Wm"], D, D)
        wsx = _pad_w(p["Wself"][:d_in], D, D)
        wsa = _pad_w(p["Wself"][d_in:], 8, D)
        fc2 = _pad_w(p["fc2"], 8, D)
        we = _pad_w(p["We"], 9, D)

        y2, sc = _node_transform(x, na, wm, wsx, wsa)
        coef = _edge_coef(edge_length_embedding, edge_attr, p["fc1"], fc2, we)
        agg = _sc_edge(y2, coef, esrc, edst)
        is_last = li == len(params) - 1
        act = (li < n_edge_layers - 1) or (li == n_edge_layers)
        x = _combine(sc, agg, act and not is_last)

    atoms_fec = x[:N, :10]

    batch_pad = jnp.pad(batch.astype(jnp.int32), (0, NPAD - N),
                        constant_values=NG - 1)
    batch3 = batch_pad.reshape(NPAD // BN, 1, BN)
    u = jax.random.uniform(jax.random.key(7), (NG, 10), jnp.float32, 1e-10, 1.0)
    g = -jnp.log(-jnp.log(u))
    y = _pool(x, batch3, g)
    return (y, atoms_fec)


# speedup vs baseline: 2.3193x; 1.0123x over previous
import functools

import jax
import jax.numpy as jnp
import numpy as np
from jax import lax
from jax.experimental import pallas as pl
from jax.experimental.pallas import tpu as pltpu
from jax.experimental.pallas import tpu_sc as plsc

N = 50000
E = 800000
NG = 64
D = 64
DH = 32
NPAD = 51200
BN = 512
BE = 2000
CH = 200
ZB = 200
NTILES = 16
EDGES_PER_TILE = E // NTILES
ROWS_PER_TILE = NPAD // NTILES
INV_SQRT_NEIGH = 0.25


def _node_body(x_ref, na_ref, wm_ref, wsx_ref, wsa_ref, y2_ref, sc_ref):
    xb = x_ref[...]
    y2_ref[0] = jnp.dot(xb, wm_ref[0], preferred_element_type=jnp.float32)
    y2_ref[1] = jnp.dot(xb, wm_ref[1], preferred_element_type=jnp.float32)
    sc = jnp.dot(xb, wsx_ref[...], preferred_element_type=jnp.float32)
    sc += jnp.dot(na_ref[...], wsa_ref[...], preferred_element_type=jnp.float32)
    sc_ref[...] = sc


def _node_transform(x, na, wm, wsx, wsa):
    grid = (NPAD // BN,)
    return pl.pallas_call(
        _node_body,
        grid=grid,
        in_specs=[
            pl.BlockSpec((BN, D), lambda i: (i, 0)),
            pl.BlockSpec((BN, 8), lambda i: (i, 0)),
            pl.BlockSpec((2, D, DH), lambda i: (0, 0, 0)),
            pl.BlockSpec((D, D), lambda i: (0, 0)),
            pl.BlockSpec((8, D), lambda i: (0, 0)),
        ],
        out_specs=[
            pl.BlockSpec((2, BN, DH), lambda i: (0, i, 0)),
            pl.BlockSpec((BN, D), lambda i: (i, 0)),
        ],
        out_shape=[
            jax.ShapeDtypeStruct((2, NPAD, DH), jnp.float32),
            jax.ShapeDtypeStruct((NPAD, D), jnp.float32),
        ],
    )(x, na, wm, wsx, wsa)


def _edge_body(elen_ref, eattr_ref, fc1_ref, fc2_ref, we_ref, coef_ref):
    w8 = jnp.maximum(
        jnp.dot(elen_ref[...], fc1_ref[...], preferred_element_type=jnp.float32),
        0.0)
    for h in range(2):
        w = jnp.dot(w8, fc2_ref[h], preferred_element_type=jnp.float32)
        ew = jnp.dot(eattr_ref[...], we_ref[h],
                     preferred_element_type=jnp.float32)
        coef_ref[h] = w * ew


def _edge_coef(elen, eattr, fc1, fc2, we):
    grid = (E // BE,)
    return pl.pallas_call(
        _edge_body,
        grid=grid,
        in_specs=[
            pl.BlockSpec((BE, 10), lambda i: (i, 0)),
            pl.BlockSpec((BE, 9), lambda i: (i, 0)),
            pl.BlockSpec((10, 8), lambda i: (0, 0)),
            pl.BlockSpec((2, 8, DH), lambda i: (0, 0, 0)),
            pl.BlockSpec((2, 9, DH), lambda i: (0, 0, 0)),
        ],
        out_specs=pl.BlockSpec((2, BE, DH), lambda i: (0, i, 0)),
        out_shape=jax.ShapeDtypeStruct((2, E, DH), jnp.float32),
    )(elen, eattr, fc1, fc2, we)


def _combine_body(act, sc_ref, agg_ref, out_ref):
    agg = jnp.concatenate([agg_ref[0], agg_ref[1]], axis=1)
    h = sc_ref[...] + agg * INV_SQRT_NEIGH
    out_ref[...] = jax.nn.gelu(h) if act else h


def _combine(sc, agg, act):
    grid = (NPAD // BN,)
    return pl.pallas_call(
        functools.partial(_combine_body, act),
        grid=grid,
        in_specs=[
            pl.BlockSpec((BN, D), lambda i: (i, 0)),
            pl.BlockSpec((2, BN, DH), lambda i: (0, i, 0)),
        ],
        out_specs=pl.BlockSpec((BN, D), lambda i: (i, 0)),
        out_shape=jax.ShapeDtypeStruct((NPAD, D), jnp.float32),
    )(sc, agg)


def _pool_body(x_ref, b_ref, g_ref, y_ref, acc_ref):
    i = pl.program_id(0)

    @pl.when(i == 0)
    def _():
        acc_ref[...] = jnp.zeros((NG, D), jnp.float32)

    b = b_ref[0, 0, :]
    oh = (lax.broadcasted_iota(jnp.int32, (NG, BN), 0) == b[None, :])
    acc_ref[...] += jnp.dot(oh.astype(jnp.float32), x_ref[...],
                            preferred_element_type=jnp.float32)

    @pl.when(i == pl.num_programs(0) - 1)
    def _():
        acc = acc_ref[...]
        z = (acc[:, :10] + g_ref[...]) * 0.01
        m = jnp.max(z, axis=1, keepdims=True)
        e = jnp.exp(z - m)
        y_ref[...] = e / jnp.sum(e, axis=1, keepdims=True)


def _pool(x, batch3, g):
    grid = (NPAD // BN,)
    return pl.pallas_call(
        _pool_body,
        grid=grid,
        in_specs=[
            pl.BlockSpec((BN, D), lambda i: (i, 0)),
            pl.BlockSpec((1, 1, BN), lambda i: (i, 0, 0)),
            pl.BlockSpec((NG, 10), lambda i: (0, 0)),
        ],
        out_specs=pl.BlockSpec((NG, 10), lambda i: (0, 0)),
        out_shape=jax.ShapeDtypeStruct((NG, 10), jnp.float32),
        scratch_shapes=[pltpu.VMEM((NG, D), jnp.float32)],
    )(x, batch3, g)


def _sc_body(y_hbm, coef_hbm, esrc_hbm, edst_hbm, out_hbm,
             src_a, dst_a, rows_a, coef_a, src_b, dst_b, rows_b, coef_b,
             agg_sh, sem_fa, sem_fb, sem_ga, sem_gb):
    c = lax.axis_index("c")
    s = lax.axis_index("s")
    ebase = s * EDGES_PER_TILE
    base_r = s * ROWS_PER_TILE

    def zrow(r, carry):
        rows_a[r, pl.ds(0, 16)] = jnp.zeros((16,), jnp.float32)
        rows_a[r, pl.ds(16, 16)] = jnp.zeros((16,), jnp.float32)
        return carry

    lax.fori_loop(0, ZB, zrow, 0)

    def zcp(k, carry):
        pltpu.sync_copy(rows_a.at[pl.ds(0, ZB)],
                        agg_sh.at[pl.ds(base_r + k * ZB, ZB)])
        return carry

    lax.fori_loop(0, ROWS_PER_TILE // ZB, zcp, 0)
    plsc.subcore_barrier()

    def fetch(i, src_v, dst_v, coef_v, sem):
        off = ebase + i * CH
        pltpu.async_copy(esrc_hbm.at[pl.ds(off, CH)], src_v, sem)
        pltpu.async_copy(edst_hbm.at[pl.ds(off, CH)], dst_v, sem)
        pltpu.async_copy(coef_hbm.at[c].at[pl.ds(off, CH)], coef_v, sem)

    def wait_fetch(src_v, dst_v, coef_v, sem):
        pltpu.make_async_copy(esrc_hbm.at[pl.ds(0, CH)], src_v, sem).wait()
        pltpu.make_async_copy(edst_hbm.at[pl.ds(0, CH)], dst_v, sem).wait()
        pltpu.make_async_copy(coef_hbm.at[c].at[pl.ds(0, CH)], coef_v,
                              sem).wait()

    def gather(src_v, rows_v, sem):
        pltpu.async_copy(y_hbm.at[c].at[src_v], rows_v, sem)

    def wait_gather(src_v, rows_v, sem):
        pltpu.make_async_copy(y_hbm.at[c].at[src_v], rows_v, sem).wait()

    def mul_scatter(dst_v, rows_v, coef_v):
        @plsc.parallel_loop(0, CH, unroll=4)
        def _(r):
            rows_v[r, pl.ds(0, 16)] = (rows_v[r, pl.ds(0, 16)]
                                       * coef_v[r, pl.ds(0, 16)])
            rows_v[r, pl.ds(16, 16)] = (rows_v[r, pl.ds(16, 16)]
                                        * coef_v[r, pl.ds(16, 16)])

        pltpu.sync_copy(rows_v, agg_sh.at[dst_v], add=True)

    bufa = (src_a, dst_a, rows_a, coef_a, sem_fa, sem_ga)
    bufb = (src_b, dst_b, rows_b, coef_b, sem_fb, sem_gb)

    fetch(0, src_a, dst_a, coef_a, sem_fa)
    wait_fetch(src_a, dst_a, coef_a, sem_fa)
    gather(src_a, rows_a, sem_ga)

    npairs = EDGES_PER_TILE // CH // 2

    def pair(i2, carry):
        def half(iA, iB, A, B, last):
            src_A, dst_A, rows_A, coef_A, sfA, sgA = A
            src_B, dst_B, rows_B, coef_B, sfB, sgB = B
            fetch(iB, src_B, dst_B, coef_B, sfB)
            wait_gather(src_A, rows_A, sgA)
            wait_fetch(src_B, dst_B, coef_B, sfB)
            gather(src_B, rows_B, sgB)
            mul_scatter(dst_A, rows_A, coef_A)

        half(2 * i2, 2 * i2 + 1, bufa, bufb, False)

        @pl.when(i2 < npairs - 1)
        def _():
            half(2 * i2 + 1, 2 * i2 + 2, bufb, bufa, False)

        @pl.when(i2 == npairs - 1)
        def _():
            wait_gather(src_b, rows_b, sem_gb)
            mul_scatter(dst_b, rows_b, coef_b)

        return carry

    lax.fori_loop(0, npairs, pair, 0)
    plsc.subcore_barrier()

    def fcp(k, carry):
        off = base_r + k * ZB
        pltpu.sync_copy(agg_sh.at[pl.ds(off, ZB)],
                        out_hbm.at[c].at[pl.ds(off, ZB)])
        return carry

    lax.fori_loop(0, ROWS_PER_TILE // ZB, fcp, 0)


@functools.lru_cache(maxsize=None)
def _sc_edge_kernel():
    mesh = plsc.VectorSubcoreMesh(core_axis_name="c", subcore_axis_name="s")
    return pl.kernel(
        _sc_body,
        out_type=jax.ShapeDtypeStruct((2, NPAD, DH), jnp.float32),
        mesh=mesh,
        scratch_types=[
            pltpu.VMEM((CH,), jnp.int32),
            pltpu.VMEM((CH,), jnp.int32),
            pltpu.VMEM((CH, DH), jnp.float32),
            pltpu.VMEM((CH, DH), jnp.float32),
            pltpu.VMEM((CH,), jnp.int32),
            pltpu.VMEM((CH,), jnp.int32),
            pltpu.VMEM((CH, DH), jnp.float32),
            pltpu.VMEM((CH, DH), jnp.float32),
            pltpu.VMEM_SHARED((NPAD, DH), jnp.float32),
            pltpu.SemaphoreType.DMA,
            pltpu.SemaphoreType.DMA,
            pltpu.SemaphoreType.DMA,
            pltpu.SemaphoreType.DMA,
        ],
        compiler_params=pltpu.CompilerParams(use_tc_tiling_on_sc=False),
    )


def _sc_edge(y2, coef, esrc, edst):
    return _sc_edge_kernel()(y2, coef, esrc, edst)


def _pad_w(w, rows, cols):
    return jnp.pad(w, ((0, rows - w.shape[0]), (0, cols - w.shape[1])))


def kernel(embed_on_edges, node_attr, edge_src, edge_dst, edge_attr,
           edge_length_embedding, batch, params_edges, params_final):
    x = jnp.pad(embed_on_edges,
                ((0, NPAD - N), (0, D - embed_on_edges.shape[1])))
    na = jnp.pad(node_attr, ((0, NPAD - N), (0, 0)))
    esrc = edge_src.astype(jnp.int32)
    edst = edge_dst.astype(jnp.int32)

    params = list(params_edges) + list(params_final)
    n_edge_layers = len(params_edges)
    d_ins = [embed_on_edges.shape[1]] + [36] * (len(params) - 1)

    for li, p in enumerate(params):
        d_in = d_ins[li]
        wm = _pad_w(p["Wm"], D, D).reshape(D, 2, DH).transpose(1, 0, 2)
        wsx = _pad_w(p["Wself"][:d_in], D, D)
        wsa = _pad_w(p["Wself"][d_in:], 8, D)
        fc2 = _pad_w(p["fc2"], 8, D).reshape(8, 2, DH).transpose(1, 0, 2)
        we = _pad_w(p["We"], 9, D).reshape(9, 2, DH).transpose(1, 0, 2)

        y2, sc = _node_transform(x, na, wm, wsx, wsa)
        coef = _edge_coef(edge_length_embedding, edge_attr, p["fc1"], fc2, we)
        agg = _sc_edge(y2, coef, esrc, edst)
        is_last = li == len(params) - 1
        act = (li < n_edge_layers - 1) or (li == n_edge_layers)
        x = _combine(sc, agg, act and not is_last)

    atoms_fec = x[:N, :10]

    batch_pad = jnp.pad(batch.astype(jnp.int32), (0, NPAD - N),
                        constant_values=NG - 1)
    batch3 = batch_pad.reshape(NPAD // BN, 1, BN)
    u = jax.random.uniform(jax.random.key(7), (NG, 10), jnp.float32, 1e-10, 1.0)
    g = -jnp.log(-jnp.log(u))
    y = _pool(x, batch3, g)
    return (y, atoms_fec)
